# baseline (speedup 1.0000x reference)
import functools

import jax
import jax.numpy as jnp
from jax.experimental import pallas as pl

HID = 128
EDGE_BLK = 4000


def _bdot(a, b):
    return jnp.dot(a.astype(jnp.bfloat16), b.astype(jnp.bfloat16),
                   preferred_element_type=jnp.float32,
                   precision=jax.lax.Precision.HIGHEST)


def _bdot_tc(a, b):
    return jnp.dot(a.astype(jnp.bfloat16), b.astype(jnp.bfloat16),
                   preferred_element_type=jnp.float32)


def _edge_block_body(inp_ref, W1_ref, b1_ref, W2_ref, b2_ref,
                     Wc1_ref, bc1_ref, wc2_ref, bc2_ref, m_ref, coord_ref):
    u = jax.nn.silu(_bdot_tc(inp_ref[...], W1_ref[...]) + b1_ref[...])
    m = jax.nn.silu(_bdot_tc(u, W2_ref[...]) + b2_ref[...])
    c1 = jax.nn.silu(_bdot_tc(m, Wc1_ref[...]) + bc1_ref[...])
    coord = _bdot_tc(c1, wc2_ref[...]) + bc2_ref[...]
    m_ref[...] = m
    coord_ref[...] = coord


@functools.partial(jax.jit, static_argnames=("n_edges",))
def _edge_mlp(inp, W1, b1, W2, b2, Wc1, bc1, wc2, bc2, n_edges):
    nblk = n_edges // EDGE_BLK
    eb = lambda w: pl.BlockSpec((EDGE_BLK, w), lambda i: (i, 0))
    fullb = lambda a: pl.BlockSpec(a.shape, lambda i: (0, 0))
    wargs = (W1, b1, W2, b2, Wc1, bc1, wc2, bc2)
    return pl.pallas_call(
        _edge_block_body,
        grid=(nblk,),
        in_specs=[eb(inp.shape[1])] + [fullb(a) for a in wargs],
        out_specs=[eb(HID), eb(1)],
        out_shape=[jax.ShapeDtypeStruct((n_edges, HID), jnp.float32),
                   jax.ShapeDtypeStruct((n_edges, 1), jnp.float32)],
    )(inp, W1, b1, W2, b2, Wc1, bc1, wc2, bc2)


def kernel(x, h, edge_fea, params, edge_index):
    row, col = edge_index[0], edge_index[1]
    E = row.shape[0]
    h = _bdot(h, params["embed"]["w"]) + params["embed"]["b"]
    deg = jnp.zeros((x.shape[0], 3), jnp.float32).at[row].add(1.0)
    deg_clip = jnp.clip(deg, 1.0, None)
    n_layers = len(params["layers"])
    for li, p in enumerate(params["layers"]):
        rij = x[row] - x[col]
        scal = jnp.sum(rij * rij, axis=-1, keepdims=True)
        inp = jnp.concatenate([scal, h[row], h[col], edge_fea], axis=-1)
        m, coord = _edge_mlp(
            inp, p["edge_mlp"][0]["w"], p["edge_mlp"][0]["b"].reshape(1, -1),
            p["edge_mlp"][1]["w"], p["edge_mlp"][1]["b"].reshape(1, -1),
            p["coord_mlp"][0]["w"], p["coord_mlp"][0]["b"].reshape(1, -1),
            p["coord_mlp"][1]["w"], p["coord_mlp"][1]["b"].reshape(1, 1),
            n_edges=E)
        f = rij * coord
        tot_f = jnp.zeros((x.shape[0], 3), jnp.float32).at[row].add(f)
        tot_f = jnp.clip(tot_f / deg_clip, -100.0, 100.0)
        x = x + tot_f
        if li + 1 < n_layers:
            tot_m = jnp.zeros((x.shape[0], HID), jnp.float32).at[row].add(m)
            node_message = jnp.concatenate([h, tot_m], axis=-1)
            un = jax.nn.silu(_bdot(node_message, p["node_mlp"][0]["w"]) + p["node_mlp"][0]["b"])
            h = _bdot(un, p["node_mlp"][1]["w"]) + p["node_mlp"][1]["b"]
    return x


# speedup vs baseline: 1.1328x; 1.1328x over previous
import functools

import jax
import jax.numpy as jnp
from jax.experimental import pallas as pl

HID = 128
EDGE_BLK = 4000


def _bdot(a, b):
    return jnp.dot(a.astype(jnp.bfloat16), b.astype(jnp.bfloat16),
                   preferred_element_type=jnp.float32,
                   precision=jax.lax.Precision.HIGHEST)


def _bdot_tc(a, b):
    return jnp.dot(a.astype(jnp.bfloat16), b.astype(jnp.bfloat16),
                   preferred_element_type=jnp.float32)


def _edge_block_body(rij_ref, hr_ref, hc_ref, ef_ref,
                     W1_ref, b1_ref, W2_ref, b2_ref,
                     Wc1_ref, bc1_ref, wc2_ref, bc2_ref, m_ref, f_ref):
    rij = rij_ref[...]
    scal = jnp.sum(rij * rij, axis=-1, keepdims=True)
    inp = jnp.concatenate([scal, hr_ref[...], hc_ref[...], ef_ref[...]],
                          axis=-1)
    u = jax.nn.silu(_bdot_tc(inp, W1_ref[...]) + b1_ref[...])
    m = jax.nn.silu(_bdot_tc(u, W2_ref[...]) + b2_ref[...])
    c1 = jax.nn.silu(_bdot_tc(m, Wc1_ref[...]) + bc1_ref[...])
    coord = _bdot_tc(c1, wc2_ref[...]) + bc2_ref[...]
    m_ref[...] = m
    f_ref[...] = rij * coord


@functools.partial(jax.jit, static_argnames=("n_edges",))
def _edge_mlp(rij, hr, hc, ef, W1, b1, W2, b2, Wc1, bc1, wc2, bc2, n_edges):
    nblk = n_edges // EDGE_BLK
    eb = lambda w: pl.BlockSpec((EDGE_BLK, w), lambda i: (i, 0))
    fullb = lambda a: pl.BlockSpec(a.shape, lambda i: (0, 0))
    wargs = (W1, b1, W2, b2, Wc1, bc1, wc2, bc2)
    return pl.pallas_call(
        _edge_block_body,
        grid=(nblk,),
        in_specs=[eb(3), eb(HID), eb(HID), eb(16)] + [fullb(a) for a in wargs],
        out_specs=[eb(HID), eb(3)],
        out_shape=[jax.ShapeDtypeStruct((n_edges, HID), jnp.float32),
                   jax.ShapeDtypeStruct((n_edges, 3), jnp.float32)],
    )(rij, hr, hc, ef, W1, b1, W2, b2, Wc1, bc1, wc2, bc2)


def kernel(x, h, edge_fea, params, edge_index):
    row, col = edge_index[0], edge_index[1]
    E = row.shape[0]
    h = _bdot(h, params["embed"]["w"]) + params["embed"]["b"]
    deg = jnp.zeros((x.shape[0], 3), jnp.float32).at[row].add(1.0)
    deg_clip = jnp.clip(deg, 1.0, None)
    n_layers = len(params["layers"])
    for li, p in enumerate(params["layers"]):
        rij = x[row] - x[col]
        m, f = _edge_mlp(
            rij, h[row], h[col], edge_fea,
            p["edge_mlp"][0]["w"], p["edge_mlp"][0]["b"].reshape(1, -1),
            p["edge_mlp"][1]["w"], p["edge_mlp"][1]["b"].reshape(1, -1),
            p["coord_mlp"][0]["w"], p["coord_mlp"][0]["b"].reshape(1, -1),
            p["coord_mlp"][1]["w"], p["coord_mlp"][1]["b"].reshape(1, 1),
            n_edges=E)
        tot_f = jnp.zeros((x.shape[0], 3), jnp.float32).at[row].add(f)
        tot_f = jnp.clip(tot_f / deg_clip, -100.0, 100.0)
        x = x + tot_f
        if li + 1 < n_layers:
            tot_m = jnp.zeros((x.shape[0], HID), jnp.float32).at[row].add(m)
            node_message = jnp.concatenate([h, tot_m], axis=-1)
            un = jax.nn.silu(_bdot(node_message, p["node_mlp"][0]["w"]) + p["node_mlp"][0]["b"])
            h = _bdot(un, p["node_mlp"][1]["w"]) + p["node_mlp"][1]["b"]
    return x


# speedup vs baseline: 1.2161x; 1.0736x over previous
import functools

import jax
import jax.numpy as jnp
from jax.experimental import pallas as pl

HID = 128
EDGE_BLK = 4000


def _bdot(a, b):
    return jnp.dot(a.astype(jnp.bfloat16), b.astype(jnp.bfloat16),
                   preferred_element_type=jnp.float32,
                   precision=jax.lax.Precision.HIGHEST)


def _bdot_tc(a, b):
    return jnp.dot(a.astype(jnp.bfloat16), b.astype(jnp.bfloat16),
                   preferred_element_type=jnp.float32)


def _edge_block_body(rij_ref, hr_ref, hc_ref, ef_ref,
                     W1_ref, b1_ref, W2_ref, b2_ref,
                     Wc1_ref, bc1_ref, wc2_ref, bc2_ref, m_ref, f_ref):
    rij = rij_ref[...]
    scal = jnp.sum(rij * rij, axis=-1, keepdims=True)
    inp = jnp.concatenate([scal, hr_ref[...], hc_ref[...], ef_ref[...]],
                          axis=-1)
    u = jax.nn.silu(_bdot_tc(inp, W1_ref[...]) + b1_ref[...])
    m = jax.nn.silu(_bdot_tc(u, W2_ref[...]) + b2_ref[...])
    c1 = jax.nn.silu(_bdot_tc(m, Wc1_ref[...]) + bc1_ref[...])
    coord = _bdot_tc(c1, wc2_ref[...]) + bc2_ref[...]
    m_ref[...] = m
    f_ref[...] = rij * coord


@functools.partial(jax.jit, static_argnames=("n_edges",))
def _edge_mlp(rij, hr, hc, ef, W1, b1, W2, b2, Wc1, bc1, wc2, bc2, n_edges):
    nblk = n_edges // EDGE_BLK
    eb = lambda w: pl.BlockSpec((EDGE_BLK, w), lambda i: (i, 0))
    fullb = lambda a: pl.BlockSpec(a.shape, lambda i: (0, 0))
    wargs = (W1, b1, W2, b2, Wc1, bc1, wc2, bc2)
    return pl.pallas_call(
        _edge_block_body,
        grid=(nblk,),
        in_specs=[eb(3), eb(HID), eb(HID), eb(16)] + [fullb(a) for a in wargs],
        out_specs=[eb(HID), eb(3)],
        out_shape=[jax.ShapeDtypeStruct((n_edges, HID), jnp.float32),
                   jax.ShapeDtypeStruct((n_edges, 3), jnp.float32)],
    )(rij, hr, hc, ef, W1, b1, W2, b2, Wc1, bc1, wc2, bc2)


def kernel(x, h, edge_fea, params, edge_index):
    row, col = edge_index[0], edge_index[1]
    E = row.shape[0]
    perm = jnp.argsort(row, stable=True)
    row = row[perm]
    col = col[perm]
    edge_fea = edge_fea[perm]
    h = _bdot(h, params["embed"]["w"]) + params["embed"]["b"]
    deg = jnp.zeros((x.shape[0], 3), jnp.float32).at[row].add(
        1.0, indices_are_sorted=True)
    deg_clip = jnp.clip(deg, 1.0, None)
    n_layers = len(params["layers"])
    for li, p in enumerate(params["layers"]):
        rij = x[row] - x[col]
        m, f = _edge_mlp(
            rij, h[row], h[col], edge_fea,
            p["edge_mlp"][0]["w"], p["edge_mlp"][0]["b"].reshape(1, -1),
            p["edge_mlp"][1]["w"], p["edge_mlp"][1]["b"].reshape(1, -1),
            p["coord_mlp"][0]["w"], p["coord_mlp"][0]["b"].reshape(1, -1),
            p["coord_mlp"][1]["w"], p["coord_mlp"][1]["b"].reshape(1, 1),
            n_edges=E)
        tot_f = jnp.zeros((x.shape[0], 3), jnp.float32).at[row].add(
            f, indices_are_sorted=True)
        tot_f = jnp.clip(tot_f / deg_clip, -100.0, 100.0)
        x = x + tot_f
        if li + 1 < n_layers:
            tot_m = jnp.zeros((x.shape[0], HID), jnp.float32).at[row].add(
                m, indices_are_sorted=True)
            node_message = jnp.concatenate([h, tot_m], axis=-1)
            un = jax.nn.silu(_bdot(node_message, p["node_mlp"][0]["w"]) + p["node_mlp"][0]["b"])
            h = _bdot(un, p["node_mlp"][1]["w"]) + p["node_mlp"][1]["b"]
    return x


# speedup vs baseline: 1.3843x; 1.1383x over previous
import functools

import jax
import jax.numpy as jnp
from jax.experimental import pallas as pl
from jax.experimental.pallas import tpu as pltpu
from jax.experimental.pallas import tpu_sc as plsc

HID = 128
EDGE_BLK = 4000
_NC, _NS = 2, 16
_NW = _NC * _NS


def _bdot(a, b):
    return jnp.dot(a.astype(jnp.bfloat16), b.astype(jnp.bfloat16),
                   preferred_element_type=jnp.float32,
                   precision=jax.lax.Precision.HIGHEST)


def _bdot_tc(a, b):
    return jnp.dot(a.astype(jnp.bfloat16), b.astype(jnp.bfloat16),
                   preferred_element_type=jnp.float32)


def _edge_block_body(with_m, rij_ref, hr_ref, hc_ref, ef_ref,
                     W1_ref, b1_ref, W2_ref, b2_ref,
                     Wc1_ref, bc1_ref, wc2_ref, bc2_ref, *out_ref):
    if with_m:
        pass
    else:
        (out_ref,) = out_ref
    rij = rij_ref[...]
    scal = jnp.sum(rij * rij, axis=-1, keepdims=True)
    inp = jnp.concatenate([scal, hr_ref[...], hc_ref[...], ef_ref[...]],
                          axis=-1)
    u = jax.nn.silu(_bdot_tc(inp, W1_ref[...]) + b1_ref[...])
    m = jax.nn.silu(_bdot_tc(u, W2_ref[...]) + b2_ref[...])
    c1 = jax.nn.silu(_bdot_tc(m, Wc1_ref[...]) + bc1_ref[...])
    coord = _bdot_tc(c1, wc2_ref[...]) + bc2_ref[...]
    f = rij * coord
    ones = jnp.ones((f.shape[0], 3), jnp.float32)
    pad = jnp.zeros((f.shape[0], 122), jnp.float32)
    if with_m:
        m_ref, f_ref = out_ref
        m_ref[...] = m
        f_ref[...] = jnp.concatenate([f, ones, pad], axis=-1)
    else:
        out_ref[...] = jnp.concatenate([f, ones, pad], axis=-1)


@functools.partial(jax.jit, static_argnames=("n_edges", "with_m"))
def _edge_mlp(rij, hr, hc, ef, W1, b1, W2, b2, Wc1, bc1, wc2, bc2,
              n_edges, with_m):
    nblk = n_edges // EDGE_BLK
    eb = lambda w: pl.BlockSpec((EDGE_BLK, w), lambda i: (i, 0))
    fullb = lambda a: pl.BlockSpec(a.shape, lambda i: (0, 0))
    wargs = (W1, b1, W2, b2, Wc1, bc1, wc2, bc2)
    oshape = jax.ShapeDtypeStruct((n_edges, HID), jnp.float32)
    out_specs = [eb(HID), eb(HID)] if with_m else [eb(HID)]
    out_shape = [oshape, oshape] if with_m else [oshape]
    return pl.pallas_call(
        functools.partial(_edge_block_body, with_m),
        grid=(nblk,),
        in_specs=[eb(3), eb(HID), eb(HID), eb(16)] + [fullb(a) for a in wargs],
        out_specs=out_specs,
        out_shape=out_shape,
    )(rij, hr, hc, ef, W1, b1, W2, b2, Wc1, bc1, wc2, bc2)


def _make_sc_scatter(E, N, D):
    W = 256
    NWIN = E // W
    ROWS_T = N // _NS
    mesh = plsc.VectorSubcoreMesh(core_axis_name="c", subcore_axis_name="s",
                                  num_cores=_NC, num_subcores=_NS)

    def body(upd_hbm, idx_hbm, zero_hbm, out_hbm, win_v, idx_v, acc_sh):
        c = jax.lax.axis_index("c")
        s = jax.lax.axis_index("s")
        wid = c * _NS + s
        pltpu.sync_copy(zero_hbm, acc_sh.at[pl.ds(s * ROWS_T, ROWS_T)])
        plsc.subcore_barrier()
        lo = (wid * NWIN) // _NW
        hi = ((wid + 1) * NWIN) // _NW

        def step(w, carry):
            pltpu.sync_copy(idx_hbm.at[pl.ds(w * W, W)], idx_v)
            pltpu.sync_copy(upd_hbm.at[pl.ds(w * W, W)], win_v)
            pltpu.sync_copy(win_v, acc_sh.at[idx_v], add=True)
            return carry

        jax.lax.fori_loop(lo, hi, step, 0)
        plsc.subcore_barrier()
        pltpu.sync_copy(acc_sh.at[pl.ds(s * ROWS_T, ROWS_T)],
                        out_hbm.at[c, pl.ds(s * ROWS_T, ROWS_T)])

    return pl.kernel(
        body,
        out_type=jax.ShapeDtypeStruct((_NC, N, D), jnp.float32),
        mesh=mesh,
        scratch_types=[
            pltpu.VMEM((W, D), jnp.float32),
            pltpu.VMEM((W,), jnp.int32),
            pltpu.VMEM_SHARED((N, D), jnp.float32),
        ],
    )


@functools.partial(jax.jit, static_argnames=("n_nodes",))
def _sc_scatter(upd, idx1d, n_nodes):
    E, D = upd.shape
    npad = -(-n_nodes // (8 * _NS)) * (8 * _NS)
    zeros_t = jnp.zeros((npad // _NS, D), jnp.float32)
    part = _make_sc_scatter(E, npad, D)(upd, idx1d, zeros_t)
    return (part[0] + part[1])[:n_nodes]


def kernel(x, h, edge_fea, params, edge_index):
    row, col = edge_index[0], edge_index[1]
    E = row.shape[0]
    N = x.shape[0]
    perm = jnp.argsort(row, stable=True)
    row = row[perm]
    col = col[perm]
    edge_fea = edge_fea[perm]
    idx1d = row
    h = _bdot(h, params["embed"]["w"]) + params["embed"]["b"]
    deg_clip = None
    n_layers = len(params["layers"])
    for li, p in enumerate(params["layers"]):
        with_m = li + 1 < n_layers
        rij = x[row] - x[col]
        outs = _edge_mlp(
            rij, h[row], h[col], edge_fea,
            p["edge_mlp"][0]["w"], p["edge_mlp"][0]["b"].reshape(1, -1),
            p["edge_mlp"][1]["w"], p["edge_mlp"][1]["b"].reshape(1, -1),
            p["coord_mlp"][0]["w"], p["coord_mlp"][0]["b"].reshape(1, -1),
            p["coord_mlp"][1]["w"], p["coord_mlp"][1]["b"].reshape(1, 1),
            n_edges=E, with_m=with_m)
        fpack = outs[-1]
        totf = _sc_scatter(fpack, idx1d, n_nodes=N)
        if deg_clip is None:
            deg_clip = jnp.clip(totf[:, 3:6], 1.0, None)
        tot_f = jnp.clip(totf[:, :3] / deg_clip, -100.0, 100.0)
        x = x + tot_f
        if with_m:
            tot_m = _sc_scatter(outs[0], idx1d, n_nodes=N)
            node_message = jnp.concatenate([h, tot_m], axis=-1)
            un = jax.nn.silu(_bdot(node_message, p["node_mlp"][0]["w"]) + p["node_mlp"][0]["b"])
            h = _bdot(un, p["node_mlp"][1]["w"]) + p["node_mlp"][1]["b"]
    return x


# speedup vs baseline: 2.0448x; 1.4772x over previous
import functools

import jax
import jax.numpy as jnp
from jax.experimental import pallas as pl
from jax.experimental.pallas import tpu as pltpu
from jax.experimental.pallas import tpu_sc as plsc

HID = 128
EDGE_BLK = 4000
_NC, _NS = 2, 16
_NW = _NC * _NS


def _bdot(a, b):
    return jnp.dot(a.astype(jnp.bfloat16), b.astype(jnp.bfloat16),
                   preferred_element_type=jnp.float32,
                   precision=jax.lax.Precision.HIGHEST)


def _bdot_tc(a, b):
    return jnp.dot(a.astype(jnp.bfloat16), b.astype(jnp.bfloat16),
                   preferred_element_type=jnp.float32)


def _edge_block_body(with_m, rij_ref, hr_ref, hc_ref, ef_ref,
                     W1_ref, b1_ref, W2_ref, b2_ref,
                     Wc1_ref, bc1_ref, wc2_ref, bc2_ref, *out_ref):
    if with_m:
        pass
    else:
        (out_ref,) = out_ref
    rij = rij_ref[...]
    scal = jnp.sum(rij * rij, axis=-1, keepdims=True)
    inp = jnp.concatenate([scal, hr_ref[...], hc_ref[...], ef_ref[...]],
                          axis=-1)
    u = jax.nn.silu(_bdot_tc(inp, W1_ref[...]) + b1_ref[...])
    m = jax.nn.silu(_bdot_tc(u, W2_ref[...]) + b2_ref[...])
    c1 = jax.nn.silu(_bdot_tc(m, Wc1_ref[...]) + bc1_ref[...])
    coord = _bdot_tc(c1, wc2_ref[...]) + bc2_ref[...]
    f = rij * coord
    ones = jnp.ones((f.shape[0], 3), jnp.float32)
    pad = jnp.zeros((f.shape[0], 122), jnp.float32)
    if with_m:
        m_ref, f_ref = out_ref
        m_ref[...] = m
        f_ref[...] = jnp.concatenate([f, ones, pad], axis=-1)
    else:
        out_ref[...] = jnp.concatenate([f, ones, pad], axis=-1)


@functools.partial(jax.jit, static_argnames=("n_edges", "with_m"))
def _edge_mlp(rij, hr, hc, ef, W1, b1, W2, b2, Wc1, bc1, wc2, bc2,
              n_edges, with_m):
    nblk = n_edges // EDGE_BLK
    eb = lambda w: pl.BlockSpec((EDGE_BLK, w), lambda i: (i, 0))
    fullb = lambda a: pl.BlockSpec(a.shape, lambda i: (0, 0))
    wargs = (W1, b1, W2, b2, Wc1, bc1, wc2, bc2)
    oshape = jax.ShapeDtypeStruct((n_edges, HID), jnp.float32)
    out_specs = [eb(HID), eb(HID)] if with_m else [eb(HID)]
    out_shape = [oshape, oshape] if with_m else [oshape]
    return pl.pallas_call(
        functools.partial(_edge_block_body, with_m),
        grid=(nblk,),
        in_specs=[eb(3), eb(HID), eb(HID), eb(16)] + [fullb(a) for a in wargs],
        out_specs=out_specs,
        out_shape=out_shape,
    )(rij, hr, hc, ef, W1, b1, W2, b2, Wc1, bc1, wc2, bc2)


def _make_sc_scatter(E, N, D):
    W = 256
    NWIN = E // W
    ROWS_T = N // _NS
    mesh = plsc.VectorSubcoreMesh(core_axis_name="c", subcore_axis_name="s",
                                  num_cores=_NC, num_subcores=_NS)

    def body(upd_hbm, idx_hbm, zero_hbm, out_hbm, win_v, idx_v, acc_sh):
        c = jax.lax.axis_index("c")
        s = jax.lax.axis_index("s")
        wid = c * _NS + s
        pltpu.sync_copy(zero_hbm, acc_sh.at[pl.ds(s * ROWS_T, ROWS_T)])
        plsc.subcore_barrier()
        lo = (wid * NWIN) // _NW
        hi = ((wid + 1) * NWIN) // _NW

        def step(w, carry):
            pltpu.sync_copy(idx_hbm.at[pl.ds(w * W, W)], idx_v)
            pltpu.sync_copy(upd_hbm.at[pl.ds(w * W, W)], win_v)
            pltpu.sync_copy(win_v, acc_sh.at[idx_v], add=True)
            return carry

        jax.lax.fori_loop(lo, hi, step, 0)
        plsc.subcore_barrier()
        pltpu.sync_copy(acc_sh.at[pl.ds(s * ROWS_T, ROWS_T)],
                        out_hbm.at[c, pl.ds(s * ROWS_T, ROWS_T)])

    return pl.kernel(
        body,
        out_type=jax.ShapeDtypeStruct((_NC, N, D), jnp.float32),
        mesh=mesh,
        scratch_types=[
            pltpu.VMEM((W, D), jnp.float32),
            pltpu.VMEM((W,), jnp.int32),
            pltpu.VMEM_SHARED((N, D), jnp.float32),
        ],
    )


def _make_sc_gather(E, NPAD, D):
    W = 256
    NWIN = E // W
    ROWS_T = NPAD // _NS
    mesh = plsc.VectorSubcoreMesh(core_axis_name="c", subcore_axis_name="s",
                                  num_cores=_NC, num_subcores=_NS)

    def body(tab_hbm, idxr_hbm, idxc_hbm, hr_hbm, hc_hbm,
             idx_v, rows_v, tab_sh, sem):
        c = jax.lax.axis_index("c")
        s = jax.lax.axis_index("s")
        wid = c * _NS + s
        pltpu.sync_copy(tab_hbm.at[pl.ds(s * ROWS_T, ROWS_T)],
                        tab_sh.at[pl.ds(s * ROWS_T, ROWS_T)])
        plsc.subcore_barrier()
        lo = (wid * NWIN) // _NW
        hi = ((wid + 1) * NWIN) // _NW

        def step(w, carry):
            for idx_hbm, out_hbm in ((idxr_hbm, hr_hbm), (idxc_hbm, hc_hbm)):
                pltpu.sync_copy(idx_hbm.at[pl.ds(w * W, W)], idx_v)
                pltpu.async_copy(tab_sh.at[idx_v], rows_v, sem).wait()
                pltpu.sync_copy(rows_v, out_hbm.at[pl.ds(w * W, W)])
            return carry

        jax.lax.fori_loop(lo, hi, step, 0)

    return pl.kernel(
        body,
        out_type=[jax.ShapeDtypeStruct((E, D), jnp.float32),
                  jax.ShapeDtypeStruct((E, D), jnp.float32)],
        mesh=mesh,
        scratch_types=[
            pltpu.VMEM((W,), jnp.int32),
            pltpu.VMEM((W, D), jnp.float32),
            pltpu.VMEM_SHARED((NPAD, D), jnp.float32),
            pltpu.SemaphoreType.DMA,
        ],
    )


@functools.partial(jax.jit, static_argnames=())
def _sc_gather(table, idxr, idxc):
    N, D = table.shape
    E = idxr.shape[0]
    npad = -(-N // (8 * _NS)) * (8 * _NS)
    tab = jnp.zeros((npad, D), jnp.float32).at[:N].set(table)
    return _make_sc_gather(E, npad, D)(tab, idxr, idxc)


@functools.partial(jax.jit, static_argnames=("n_nodes",))
def _sc_scatter(upd, idx1d, n_nodes):
    E, D = upd.shape
    npad = -(-n_nodes // (8 * _NS)) * (8 * _NS)
    zeros_t = jnp.zeros((npad // _NS, D), jnp.float32)
    part = _make_sc_scatter(E, npad, D)(upd, idx1d, zeros_t)
    return (part[0] + part[1])[:n_nodes]


def kernel(x, h, edge_fea, params, edge_index):
    row, col = edge_index[0], edge_index[1]
    E = row.shape[0]
    N = x.shape[0]
    perm = jnp.argsort(row, stable=True)
    row = row[perm]
    col = col[perm]
    edge_fea = edge_fea[perm]
    idx1d = row
    h = _bdot(h, params["embed"]["w"]) + params["embed"]["b"]
    deg_clip = None
    n_layers = len(params["layers"])
    for li, p in enumerate(params["layers"]):
        with_m = li + 1 < n_layers
        rij = x[row] - x[col]
        hr, hc = _sc_gather(h, row, col)
        outs = _edge_mlp(
            rij, hr, hc, edge_fea,
            p["edge_mlp"][0]["w"], p["edge_mlp"][0]["b"].reshape(1, -1),
            p["edge_mlp"][1]["w"], p["edge_mlp"][1]["b"].reshape(1, -1),
            p["coord_mlp"][0]["w"], p["coord_mlp"][0]["b"].reshape(1, -1),
            p["coord_mlp"][1]["w"], p["coord_mlp"][1]["b"].reshape(1, 1),
            n_edges=E, with_m=with_m)
        fpack = outs[-1]
        totf = _sc_scatter(fpack, idx1d, n_nodes=N)
        if deg_clip is None:
            deg_clip = jnp.clip(totf[:, 3:6], 1.0, None)
        tot_f = jnp.clip(totf[:, :3] / deg_clip, -100.0, 100.0)
        x = x + tot_f
        if with_m:
            tot_m = _sc_scatter(outs[0], idx1d, n_nodes=N)
            node_message = jnp.concatenate([h, tot_m], axis=-1)
            un = jax.nn.silu(_bdot(node_message, p["node_mlp"][0]["w"]) + p["node_mlp"][0]["b"])
            h = _bdot(un, p["node_mlp"][1]["w"]) + p["node_mlp"][1]["b"]
    return x
